# baseline (speedup 1.0000x reference)
import functools

import jax
import jax.numpy as jnp
from jax import lax
from jax.experimental import pallas as pl
from jax.experimental.pallas import tpu as pltpu
from jax.experimental.pallas import tpu_sc as plsc

_NUM_FIELDS = 26
_FIELD_DIM = 100000
_EMBED_DIM = 16
_BATCH = 16384

_NC, _NS, _L = 2, 16, 16
_NW = _NC * _NS
_SPW = _BATCH // _NW
_SPC = 4
_RPC = _SPC * _NUM_FIELDS
_NCH = _SPW // _SPC
_ENBUF = 4
_LCH = _SPW * _NUM_FIELDS // 128
_LBLK = _SPW // 128
_LNBUF = 4


def _fm_kernel(eidx_hbm, lidx_hbm, emb_hbm, lin_hbm, out_hbm,
               eidx_v, lidx_v, ebuf, lbuf, acc_v, fm_v,
               esems, lsems):
    wid = lax.axis_index("s") * _NC + lax.axis_index("c")

    pltpu.sync_copy(eidx_hbm.at[wid], eidx_v)
    pltpu.sync_copy(lidx_hbm.at[wid], lidx_v)

    zeros = jnp.zeros((_L,), jnp.float32)
    for i in range(_SPW // _L):
        acc_v[pl.ds(i * _L, _L)] = zeros

    for b in range(_LNBUF):
        pltpu.async_copy(lin_hbm.at[lidx_v.at[b]], lbuf.at[b], lsems[b])

    @pl.loop(0, _LCH // _LNBUF)
    def _lin_loop(o):
        for b in range(_LNBUF):
            c = o * _LNBUF + b
            pltpu.make_async_copy(
                lin_hbm.at[lidx_v.at[c]], lbuf.at[b], lsems[b]).wait()
            for i in range(128 // _L):
                sl = pl.ds(b * 128 + i * _L, _L)
                acc_v[sl] = acc_v[sl] + lbuf[b, pl.ds(i * _L, _L)]
            nxt = c + _LNBUF

            @pl.when(nxt < _LCH)
            def _():
                pltpu.async_copy(
                    lin_hbm.at[lidx_v.at[nxt]], lbuf.at[b], lsems[b])

    for b in range(_ENBUF):
        pltpu.async_copy(emb_hbm.at[eidx_v.at[b]], ebuf.at[b], esems[b])

    lane = lax.broadcasted_iota(jnp.int32, (_L,), 0)

    @pl.loop(0, _NCH // _ENBUF)
    def _emb_loop(o):
        fmvec = zeros
        for b in range(_ENBUF):
            j = o * _ENBUF + b
            pltpu.make_async_copy(
                emb_hbm.at[eidx_v.at[j]], ebuf.at[b], esems[b]).wait()
            for s in range(_SPC):
                e0 = ebuf[b, s * _NUM_FIELDS, :]
                sacc = e0
                ssq = e0 * e0
                for f in range(1, _NUM_FIELDS):
                    ef = ebuf[b, s * _NUM_FIELDS + f, :]
                    sacc = sacc + ef
                    ssq = ssq + ef * ef
                q = sacc * sacc - ssq
                for sh in (8, 4, 2, 1):
                    q = q + q[lane ^ sh]
                fmvec = jnp.where(lane == (b * _SPC + s), 0.5 * q, fmvec)
            nxt = j + _ENBUF

            @pl.when(nxt < _NCH)
            def _():
                pltpu.async_copy(
                    emb_hbm.at[eidx_v.at[nxt]], ebuf.at[b], esems[b])
        fm_v[pl.ds(o * _L, _L)] = fmvec

    for i in range(_SPW // _L):
        sl = pl.ds(i * _L, _L)
        acc_v[sl] = acc_v[sl] + fm_v[sl]
    pltpu.sync_copy(acc_v, out_hbm.at[pl.ds(wid * _SPW, _SPW)])


@jax.jit
def kernel(x, emb_table, lin_weight, bias):
    offsets = jnp.arange(_NUM_FIELDS, dtype=jnp.int32) * _FIELD_DIM
    idx = x + offsets[None, :]
    idx_w = idx.reshape(_NW, _SPW, _NUM_FIELDS)
    eidx = idx_w.reshape(_NW, _NCH, _RPC)
    lidx = jnp.transpose(idx_w, (0, 2, 1)).reshape(_NW, _LCH, 128)
    lin_flat = lin_weight.reshape(-1)

    mesh = plsc.VectorSubcoreMesh(core_axis_name="c", subcore_axis_name="s")
    run = pl.kernel(
        _fm_kernel,
        out_type=jax.ShapeDtypeStruct((_BATCH,), jnp.float32),
        mesh=mesh,
        compiler_params=pltpu.CompilerParams(use_tc_tiling_on_sc=False),
        scratch_types=[
            pltpu.VMEM((_NCH, _RPC), jnp.int32),
            pltpu.VMEM((_LCH, 128), jnp.int32),
            pltpu.VMEM((_ENBUF, _RPC, _EMBED_DIM), jnp.float32),
            pltpu.VMEM((_LNBUF, 128), jnp.float32),
            pltpu.VMEM((_SPW,), jnp.float32),
            pltpu.VMEM((_SPW,), jnp.float32),
            [pltpu.SemaphoreType.DMA] * _ENBUF,
            [pltpu.SemaphoreType.DMA] * _LNBUF,
        ],
    )
    logits = run(eidx, lidx, emb_table, lin_flat)
    return logits + bias[0]


# speedup vs baseline: 1.0147x; 1.0147x over previous
import functools

import jax
import jax.numpy as jnp
from jax import lax
from jax.experimental import pallas as pl
from jax.experimental.pallas import tpu as pltpu
from jax.experimental.pallas import tpu_sc as plsc

_NUM_FIELDS = 26
_FIELD_DIM = 100000
_EMBED_DIM = 16
_BATCH = 16384

_NC, _NS, _L = 2, 16, 16
_NW = _NC * _NS
_SPW = _BATCH // _NW
_SPC = 4
_RPC = _SPC * _NUM_FIELDS
_NCH = _SPW // _SPC
_ENBUF = 4
_LCH = _SPW * _NUM_FIELDS // 128
_LBLK = _SPW // 128
_LNBUF = 4


def _fm_kernel(eidx_hbm, lidx_hbm, emb_hbm, lin_hbm, out_hbm,
               eidx_v, lidx_v, ebuf, lbuf, acc_v, fm_v,
               esems, lsem):
    wid = lax.axis_index("s") * _NC + lax.axis_index("c")

    pltpu.sync_copy(eidx_hbm.at[wid], eidx_v)
    pltpu.sync_copy(lidx_hbm.at[wid], lidx_v)

    zeros = jnp.zeros((_L,), jnp.float32)
    lane = lax.broadcasted_iota(jnp.int32, (_L,), 0)

    for b in range(_ENBUF):
        pltpu.async_copy(emb_hbm.at[eidx_v.at[b]], ebuf.at[b], esems[b])

    @pl.loop(0, _NCH // _ENBUF)
    def _emb_loop(o):
        for b in range(_LNBUF):
            c = o * _LNBUF + b

            @pl.when(c < _LCH)
            def _():
                pltpu.async_copy(lin_hbm.at[lidx_v.at[c]], lbuf.at[c], lsem)

        fmvec = zeros
        for b in range(_ENBUF):
            j = o * _ENBUF + b
            pltpu.make_async_copy(
                emb_hbm.at[eidx_v.at[j]], ebuf.at[b], esems[b]).wait()
            for s in range(_SPC):
                e0 = ebuf[b, s * _NUM_FIELDS, :]
                sacc = e0
                ssq = e0 * e0
                for f in range(1, _NUM_FIELDS):
                    ef = ebuf[b, s * _NUM_FIELDS + f, :]
                    sacc = sacc + ef
                    ssq = ssq + ef * ef
                q = sacc * sacc - ssq
                for sh in (8, 4, 2, 1):
                    q = q + q[lane ^ sh]
                fmvec = jnp.where(lane == (b * _SPC + s), 0.5 * q, fmvec)
            nxt = j + _ENBUF

            @pl.when(nxt < _NCH)
            def _():
                pltpu.async_copy(
                    emb_hbm.at[eidx_v.at[nxt]], ebuf.at[b], esems[b])
        fm_v[pl.ds(o * _L, _L)] = fmvec

    @pl.loop(0, _LCH)
    def _lin_drain(c):
        pltpu.make_async_copy(lin_hbm.at[lidx_v.at[c]], lbuf.at[c], lsem).wait()

    for i in range(_SPW // _L):
        acc_v[pl.ds(i * _L, _L)] = fm_v[pl.ds(i * _L, _L)]

    @pl.loop(0, _LCH)
    def _lin_acc(c):
        blk = lax.rem(c, _LBLK)
        for i in range(128 // _L):
            sl = pl.ds(blk * 128 + i * _L, _L)
            acc_v[sl] = acc_v[sl] + lbuf[c, pl.ds(i * _L, _L)]

    pltpu.sync_copy(acc_v, out_hbm.at[pl.ds(wid * _SPW, _SPW)])


@jax.jit
def kernel(x, emb_table, lin_weight, bias):
    offsets = jnp.arange(_NUM_FIELDS, dtype=jnp.int32) * _FIELD_DIM
    idx = x + offsets[None, :]
    idx_w = idx.reshape(_NW, _SPW, _NUM_FIELDS)
    eidx = idx_w.reshape(_NW, _NCH, _RPC)
    lidx = jnp.transpose(idx_w, (0, 2, 1)).reshape(_NW, _LCH, 128)
    lin_flat = lin_weight.reshape(-1)

    mesh = plsc.VectorSubcoreMesh(core_axis_name="c", subcore_axis_name="s")
    run = pl.kernel(
        _fm_kernel,
        out_type=jax.ShapeDtypeStruct((_BATCH,), jnp.float32),
        mesh=mesh,
        compiler_params=pltpu.CompilerParams(use_tc_tiling_on_sc=False),
        scratch_types=[
            pltpu.VMEM((_NCH, _RPC), jnp.int32),
            pltpu.VMEM((_LCH, 128), jnp.int32),
            pltpu.VMEM((_ENBUF, _RPC, _EMBED_DIM), jnp.float32),
            pltpu.VMEM((_LCH, 128), jnp.float32),
            pltpu.VMEM((_SPW,), jnp.float32),
            pltpu.VMEM((_SPW,), jnp.float32),
            [pltpu.SemaphoreType.DMA] * _ENBUF,
            pltpu.SemaphoreType.DMA,
        ],
    )
    logits = run(eidx, lidx, emb_table, lin_flat)
    return logits + bias[0]
